# baseline (speedup 1.0000x reference)
import functools

import jax
import jax.numpy as jnp
from jax import lax
from jax.experimental import pallas as pl
from jax.experimental.pallas import tpu as pltpu
from jax.experimental.pallas import tpu_sc as plsc

F32 = jnp.float32
I32 = jnp.int32

N_TOK, N_DIMS, N_SM, N_BR = 8192, 2048, 1024, 8
TM = 512
TMB = 512
PAD = N_TOK + N_BR * TMB
NTILES = PAD // TMB
NC, NS, L = 2, 16, 16
NW = NC * NS
TPW = N_TOK // NW
CH = 16
ZW = 128
NCH = TPW // CH


def _router_math(xb, Wsw, bsw, en, gn):
    tm = xb.shape[0]
    E = en.shape[1]
    a = jnp.maximum(xb, 0.0)
    ctrl = jnp.dot(a, Wsw, preferred_element_type=F32) + bsw
    ylog = ctrl[:, 0:E]
    zmean = ctrl[:, E:2 * E]
    zlogv = ctrl[:, 2 * E:3 * E]
    g = -jnp.log(en + 1e-20)
    gl = (ylog + g) / 1.0
    ysoft = jax.nn.softmax(gl, axis=1)
    m = jnp.max(ysoft, axis=1, keepdims=True)
    lanes = lax.broadcasted_iota(I32, (tm, E), 1)
    cand = jnp.where(ysoft == m, lanes, E)
    yidx = jnp.min(cand, axis=1, keepdims=True)
    onehot = (lanes == yidx).astype(F32)
    yhard = (onehot - ysoft) + ysoft
    z = gn * jnp.exp(zlogv / 2.0) + zmean
    return a, ylog, yidx, yhard, zmean, zlogv, z, onehot


def _router_body(x_ref, Wsw_ref, bsw_ref, en_ref, gn_ref,
                 yl_ref, yi_ref, yh_ref, zm_ref, zl_ref, zg_ref,
                 yi1_ref, rank_ref, zaux_ref, pad16_ref, eot_ref, cnt_ref):
    i = pl.program_id(0)
    nt = pl.num_programs(0)

    @pl.when(i == 0)
    def _():
        cnt_ref[...] = jnp.zeros_like(cnt_ref)

    xb = x_ref[...]
    _, ylog, yidx, yhard, zmean, zlogv, z, onehot = _router_math(
        xb, Wsw_ref[...], bsw_ref[...], en_ref[...], gn_ref[...])

    tm = xb.shape[0]
    E = onehot.shape[1]
    yl_ref[...] = jnp.transpose(ylog)
    yi_ref[...] = jnp.transpose(yidx)
    yh_ref[...] = jnp.transpose(yhard)
    zm_ref[...] = jnp.reshape(jnp.sum(onehot * zmean, axis=1), (1, tm))
    zl_ref[...] = jnp.reshape(jnp.sum(onehot * zlogv, axis=1), (1, tm))
    zg1 = jnp.sum(onehot * z, axis=1)
    zg_ref[...] = jnp.reshape(zg1, (1, tm))
    zaux_ref[...] = jnp.broadcast_to(jnp.reshape(zg1, (tm, 1)), zaux_ref.shape)
    yi1_ref[...] = jnp.reshape(jnp.transpose(yidx), (tm,))

    rows = lax.broadcasted_iota(I32, (tm, tm), 0)
    cols = lax.broadcasted_iota(I32, (tm, tm), 1)
    tri = (cols < rows).astype(F32)
    excl = jnp.dot(tri, onehot, preferred_element_type=F32)
    carry = cnt_ref[...]
    rank_ref[...] = jnp.sum(onehot * (excl + carry), axis=1).astype(I32)
    new_cnt = carry + jnp.sum(onehot, axis=0, keepdims=True)
    cnt_ref[...] = new_cnt

    @pl.when(i == nt - 1)
    def _():
        padded = jnp.floor((new_cnt + (TMB - 1)) * (1.0 / TMB)) * TMB
        tri8 = (lax.broadcasted_iota(I32, (E, E), 0)
                <= lax.broadcasted_iota(I32, (E, E), 1)).astype(F32)
        pad_end = jnp.dot(padded, tri8, preferred_element_type=F32)
        pad_off = pad_end - padded
        pad16_ref[...] = jnp.reshape(jnp.concatenate(
            (pad_off, jnp.zeros_like(pad_off)), axis=1).astype(I32), (2 * E,))
        starts = (lax.broadcasted_iota(I32, (NTILES, 1), 0) * TMB).astype(F32)
        le = (jnp.broadcast_to(pad_end, (NTILES, E)) <= starts).astype(F32)
        eot_ref[...] = jnp.clip(jnp.sum(le, axis=1), 0.0, E - 1.0).astype(I32)


def _run_router(x, W_sw, bsw2, exp_noise, gauss_noise):
    N, D = x.shape
    E = exp_noise.shape[1]
    nt = N // TM
    out_shapes = (
        jax.ShapeDtypeStruct((E, N), F32),
        jax.ShapeDtypeStruct((1, N), I32),
        jax.ShapeDtypeStruct((E, N), F32),
        jax.ShapeDtypeStruct((1, N), F32),
        jax.ShapeDtypeStruct((1, N), F32),
        jax.ShapeDtypeStruct((1, N), F32),
        jax.ShapeDtypeStruct((N,), I32),
        jax.ShapeDtypeStruct((N,), I32),
        jax.ShapeDtypeStruct((N, ZW), F32),
        jax.ShapeDtypeStruct((2 * E,), I32),
        jax.ShapeDtypeStruct((NTILES,), I32),
    )
    fn = pl.pallas_call(
        _router_body,
        grid=(nt,),
        in_specs=[
            pl.BlockSpec((TM, D), lambda i: (i, 0)),
            pl.BlockSpec((D, 3 * E), lambda i: (0, 0)),
            pl.BlockSpec((1, 3 * E), lambda i: (0, 0)),
            pl.BlockSpec((TM, E), lambda i: (i, 0)),
            pl.BlockSpec((TM, E), lambda i: (i, 0)),
        ],
        out_specs=(
            pl.BlockSpec((E, TM), lambda i: (0, i)),
            pl.BlockSpec((1, TM), lambda i: (0, i)),
            pl.BlockSpec((E, TM), lambda i: (0, i)),
            pl.BlockSpec((1, TM), lambda i: (0, i)),
            pl.BlockSpec((1, TM), lambda i: (0, i)),
            pl.BlockSpec((1, TM), lambda i: (0, i)),
            pl.BlockSpec((TM,), lambda i: (i,)),
            pl.BlockSpec((TM,), lambda i: (i,)),
            pl.BlockSpec((TM, ZW), lambda i: (i, 0)),
            pl.BlockSpec((2 * E,), lambda i: (0,)),
            pl.BlockSpec((NTILES,), lambda i: (0,)),
        ),
        out_shape=out_shapes,
        scratch_shapes=[pltpu.VMEM((1, E), F32)],
        compiler_params=pltpu.CompilerParams(dimension_semantics=("arbitrary",)),
    )
    return fn(x, W_sw, bsw2, exp_noise, gauss_noise)


def _sc_dispatch(x, zaux, y_flat, rank_flat, pad16):
    N, D = x.shape
    mesh = plsc.VectorSubcoreMesh(core_axis_name="c", subcore_axis_name="s")

    @functools.partial(
        pl.kernel, mesh=mesh,
        out_type=(
            jax.ShapeDtypeStruct((PAD, D), F32),
            jax.ShapeDtypeStruct((PAD, ZW), F32),
            jax.ShapeDtypeStruct((N,), I32),
        ),
        scratch_types=[
            pltpu.VMEM((L,), I32),
            pltpu.VMEM((TPW,), I32),
            pltpu.VMEM((TPW,), I32),
            [pltpu.VMEM((CH,), I32) for _ in range(2)],
            [pltpu.VMEM((CH, D), F32) for _ in range(2)],
            [pltpu.VMEM((CH, ZW), F32) for _ in range(2)],
            [pltpu.SemaphoreType.DMA for _ in range(2)],
            [pltpu.SemaphoreType.DMA for _ in range(2)],
        ],
    )
    def k(y_hbm, r_hbm, pad_hbm, x_hbm, zaux_hbm, xs_hbm, zs_hbm, slot_hbm,
          pad_v, yv, rv, slotv, xv, zv, semx, semz):
        wid = lax.axis_index("s") * NC + lax.axis_index("c")
        base = wid * TPW
        pltpu.sync_copy(pad_hbm, pad_v)
        pltpu.sync_copy(y_hbm.at[pl.ds(base, TPW)], yv)
        pltpu.sync_copy(r_hbm.at[pl.ds(base, TPW)], rv)
        pv = pad_v[...]
        hx = [None, None]
        hz = [None, None]
        for c in range(NCH):
            b = c % 2
            t0 = base + c * CH
            if hx[b] is not None:
                hx[b].wait()
                hz[b].wait()
            for kk in range(CH // L):
                off = c * CH + kk * L
                y16 = yv[pl.ds(off, L)]
                r16 = rv[pl.ds(off, L)]
                po = jnp.zeros((L,), I32)
                for e in range(N_BR):
                    pe = lax.squeeze(lax.slice(pv, (e,), (e + 1,)), (0,))
                    po = jnp.where(y16 == e, pe, po)
                slotv[b][pl.ds(kk * L, L)] = po + r16
            pltpu.sync_copy(x_hbm.at[pl.ds(t0, CH)], xv[b])
            pltpu.sync_copy(zaux_hbm.at[pl.ds(t0, CH)], zv[b])
            hx[b] = pltpu.async_copy(xv[b], xs_hbm.at[slotv[b]], semx[b])
            hz[b] = pltpu.async_copy(zv[b], zs_hbm.at[slotv[b]], semz[b])
            pltpu.sync_copy(slotv[b], slot_hbm.at[pl.ds(t0, CH)])
        for b in range(2):
            if hx[b] is not None:
                hx[b].wait()
                hz[b].wait()

    return k(y_flat, rank_flat, pad16, x, zaux)


def _sc_combine(buf, slot):
    D = buf.shape[1]
    N = slot.shape[0]
    mesh = plsc.VectorSubcoreMesh(core_axis_name="c", subcore_axis_name="s")

    @functools.partial(
        pl.kernel, mesh=mesh,
        out_type=jax.ShapeDtypeStruct((N, D), F32),
        scratch_types=[
            pltpu.VMEM((TPW,), I32),
            [pltpu.VMEM((CH,), I32) for _ in range(2)],
            [pltpu.VMEM((CH, D), F32) for _ in range(2)],
            [pltpu.SemaphoreType.DMA for _ in range(2)],
        ],
    )
    def k(buf_hbm, slot_hbm, out_hbm, sv, svc, bv, sem):
        wid = lax.axis_index("s") * NC + lax.axis_index("c")
        base = wid * TPW
        pltpu.sync_copy(slot_hbm.at[pl.ds(base, TPW)], sv)

        def fill_and_fire(c):
            b = c % 2
            for kk in range(CH // L):
                svc[b][pl.ds(kk * L, L)] = sv[pl.ds(c * CH + kk * L, L)]
            return pltpu.async_copy(buf_hbm.at[svc[b]], bv[b], sem[b])

        h = [None, None]
        h[0] = fill_and_fire(0)
        for c in range(NCH):
            b = c % 2
            if c + 1 < NCH:
                h[1 - b] = fill_and_fire(c + 1)
            h[b].wait()
            pltpu.sync_copy(bv[b], out_hbm.at[pl.ds(base + c * CH, CH)])

    return k(buf, slot)


def _mlp_body(eot_ref, xs_ref, zs_ref, W1_ref, b1_ref, W2_ref, b2_ref, out_ref):
    xb = xs_ref[...]
    a = jnp.maximum(xb, 0.0)
    h = jnp.maximum(jnp.dot(a, W1_ref[0], preferred_element_type=F32) + b1_ref[0], 0.0)
    sub = jnp.dot(h, W2_ref[0], preferred_element_type=F32) + b2_ref[0]
    out_ref[...] = xb + sub * zs_ref[:, 0:1]


def _run_grouped_mlp(eot, xs, zs, W1, b1r, W2, b2r):
    D, H, E = N_DIMS, N_SM, N_BR
    grid_spec = pltpu.PrefetchScalarGridSpec(
        num_scalar_prefetch=1,
        grid=(NTILES,),
        in_specs=[
            pl.BlockSpec((TMB, D), lambda i, s: (i, 0)),
            pl.BlockSpec((TMB, ZW), lambda i, s: (i, 0)),
            pl.BlockSpec((1, D, H), lambda i, s: (s[i], 0, 0)),
            pl.BlockSpec((1, 1, H), lambda i, s: (s[i], 0, 0)),
            pl.BlockSpec((1, H, D), lambda i, s: (s[i], 0, 0)),
            pl.BlockSpec((1, 1, D), lambda i, s: (s[i], 0, 0)),
        ],
        out_specs=pl.BlockSpec((TMB, D), lambda i, s: (i, 0)),
    )
    fn = pl.pallas_call(
        _mlp_body,
        grid_spec=grid_spec,
        out_shape=jax.ShapeDtypeStruct((PAD, D), F32),
        compiler_params=pltpu.CompilerParams(dimension_semantics=("arbitrary",)),
    )
    return fn(eot, xs, zs, W1, b1r, W2, b2r)


def kernel(x, W_sw, b_sw, W1, b1, W2, b2, exp_noise, gauss_noise):
    N, D = x.shape
    E = exp_noise.shape[1]
    H = W1.shape[2]
    bsw2 = b_sw.reshape(1, 3 * E)
    b1r = b1.reshape(E, 1, H)
    b2r = b2.reshape(E, 1, D)

    ylogT, yidxT, yhardT, zmT, zlT, zgT, yidx1, rank1, zaux, pad16, eot = \
        _run_router(x, W_sw, bsw2, exp_noise, gauss_noise)

    xs, zs, slot = _sc_dispatch(x, zaux, yidx1, rank1, pad16)
    buf = _run_grouped_mlp(eot, xs, zs, W1, b1r, W2, b2r)
    out1 = _sc_combine(buf, slot)

    return (out1, jnp.transpose(ylogT), jnp.transpose(yidxT),
            jnp.transpose(yhardT), jnp.transpose(zmT), jnp.transpose(zlT),
            jnp.transpose(zgT))


# speedup vs baseline: 1.1615x; 1.1615x over previous
import functools

import jax
import jax.numpy as jnp
from jax import lax
from jax.experimental import pallas as pl
from jax.experimental.pallas import tpu as pltpu
from jax.experimental.pallas import tpu_sc as plsc

F32 = jnp.float32
I32 = jnp.int32

N_TOK, N_DIMS, N_SM, N_BR = 8192, 2048, 1024, 8
TM = 512
TMB = 256
PAD = N_TOK + N_BR * TMB
NTILES = PAD // TMB
NC, NS, L = 2, 16, 16
NW = NC * NS
TPW = N_TOK // NW
CH = 16
ZW = 128
NCH = TPW // CH


def _router_body(x_ref, Wsw_ref, bsw_ref, enT_ref, gnT_ref,
                 yl_ref, yi_ref, yh_ref, zm_ref, zl_ref, zg_ref,
                 yi1_ref, rank_ref, zaux_ref, pad16_ref, eot_ref, cnt_ref):
    i = pl.program_id(0)
    nt = pl.num_programs(0)

    @pl.when(i == 0)
    def _():
        cnt_ref[...] = jnp.zeros_like(cnt_ref)

    xb = x_ref[...]
    tm = xb.shape[0]
    E = enT_ref.shape[0]
    a = jnp.maximum(xb, 0.0)
    ctrlT = lax.dot_general(Wsw_ref[...], a, (((0,), (1,)), ((), ())),
                            preferred_element_type=F32) + bsw_ref[...]
    ylogT = ctrlT[0:E]
    zmeanT = ctrlT[E:2 * E]
    zlogvT = ctrlT[2 * E:3 * E]
    gT = -jnp.log(enT_ref[...] + 1e-20)
    glT = (ylogT + gT) / 1.0
    ysoftT = jax.nn.softmax(glT, axis=0)
    m = jnp.max(ysoftT, axis=0, keepdims=True)
    lanesT = lax.broadcasted_iota(I32, (E, tm), 0)
    cand = jnp.where(ysoftT == m, lanesT, E)
    yidxT = jnp.min(cand, axis=0, keepdims=True)
    onehotT = (lanesT == yidxT).astype(F32)
    yhardT = (onehotT - ysoftT) + ysoftT
    zT = gnT_ref[...] * jnp.exp(zlogvT / 2.0) + zmeanT

    yl_ref[...] = ylogT
    yi_ref[...] = yidxT
    yh_ref[...] = yhardT
    zm_ref[...] = jnp.sum(onehotT * zmeanT, axis=0, keepdims=True)
    zl_ref[...] = jnp.sum(onehotT * zlogvT, axis=0, keepdims=True)
    zgT = jnp.sum(onehotT * zT, axis=0, keepdims=True)
    zg_ref[...] = zgT
    zaux_ref[...] = jnp.broadcast_to(jnp.transpose(zgT), zaux_ref.shape)
    yi1_ref[...] = jnp.reshape(yidxT, (tm,))

    rows = lax.broadcasted_iota(I32, (tm, tm), 0)
    cols = lax.broadcasted_iota(I32, (tm, tm), 1)
    triu = (rows < cols).astype(F32)
    exclT = jnp.dot(onehotT, triu, preferred_element_type=F32)
    carry = cnt_ref[...]
    rank_ref[...] = jnp.sum(onehotT * (exclT + carry), axis=0).astype(I32)
    ones_col = jnp.ones((tm, 1), F32)
    new_cnt = carry + jnp.dot(onehotT, ones_col, preferred_element_type=F32)
    cnt_ref[...] = new_cnt

    @pl.when(i == nt - 1)
    def _():
        padded = jnp.floor((new_cnt + (TMB - 1)) * (1.0 / TMB)) * TMB
        tri8 = (lax.broadcasted_iota(I32, (E, E), 1)
                <= lax.broadcasted_iota(I32, (E, E), 0)).astype(F32)
        pad_end = jnp.dot(tri8, padded, preferred_element_type=F32)
        pad_off = pad_end - padded
        pad16_ref[...] = jnp.reshape(jnp.concatenate(
            (pad_off, jnp.zeros_like(pad_off)), axis=0).astype(I32), (2 * E,))
        starts = (lax.broadcasted_iota(I32, (1, NTILES), 1) * TMB).astype(F32)
        le = (jnp.broadcast_to(pad_end, (E, NTILES)) <= starts).astype(F32)
        eot_ref[...] = jnp.clip(jnp.sum(le, axis=0), 0.0, E - 1.0).astype(I32)


def _run_router(x, W_sw, bsw2, exp_noise, gauss_noise):
    N, D = x.shape
    E = exp_noise.shape[0]
    nt = N // TM
    out_shapes = (
        jax.ShapeDtypeStruct((E, N), F32),
        jax.ShapeDtypeStruct((1, N), I32),
        jax.ShapeDtypeStruct((E, N), F32),
        jax.ShapeDtypeStruct((1, N), F32),
        jax.ShapeDtypeStruct((1, N), F32),
        jax.ShapeDtypeStruct((1, N), F32),
        jax.ShapeDtypeStruct((N,), I32),
        jax.ShapeDtypeStruct((N,), I32),
        jax.ShapeDtypeStruct((N, ZW), F32),
        jax.ShapeDtypeStruct((2 * E,), I32),
        jax.ShapeDtypeStruct((NTILES,), I32),
    )
    fn = pl.pallas_call(
        _router_body,
        grid=(nt,),
        in_specs=[
            pl.BlockSpec((TM, D), lambda i: (i, 0)),
            pl.BlockSpec((D, 3 * E), lambda i: (0, 0)),
            pl.BlockSpec((3 * E, 1), lambda i: (0, 0)),
            pl.BlockSpec((E, TM), lambda i: (0, i)),
            pl.BlockSpec((E, TM), lambda i: (0, i)),
        ],
        out_specs=(
            pl.BlockSpec((E, TM), lambda i: (0, i)),
            pl.BlockSpec((1, TM), lambda i: (0, i)),
            pl.BlockSpec((E, TM), lambda i: (0, i)),
            pl.BlockSpec((1, TM), lambda i: (0, i)),
            pl.BlockSpec((1, TM), lambda i: (0, i)),
            pl.BlockSpec((1, TM), lambda i: (0, i)),
            pl.BlockSpec((TM,), lambda i: (i,)),
            pl.BlockSpec((TM,), lambda i: (i,)),
            pl.BlockSpec((TM, ZW), lambda i: (i, 0)),
            pl.BlockSpec((2 * E,), lambda i: (0,)),
            pl.BlockSpec((NTILES,), lambda i: (0,)),
        ),
        out_shape=out_shapes,
        scratch_shapes=[pltpu.VMEM((E, 1), F32)],
        compiler_params=pltpu.CompilerParams(dimension_semantics=("arbitrary",)),
    )
    return fn(x, W_sw, bsw2, exp_noise, gauss_noise)


def _sc_dispatch(x, zaux, y_flat, rank_flat, pad16):
    N, D = x.shape
    mesh = plsc.VectorSubcoreMesh(core_axis_name="c", subcore_axis_name="s")

    @functools.partial(
        pl.kernel, mesh=mesh,
        out_type=(
            jax.ShapeDtypeStruct((PAD, D), F32),
            jax.ShapeDtypeStruct((PAD, ZW), F32),
            jax.ShapeDtypeStruct((N,), I32),
        ),
        scratch_types=[
            pltpu.VMEM((L,), I32),
            pltpu.VMEM((TPW,), I32),
            pltpu.VMEM((TPW,), I32),
            [pltpu.VMEM((CH,), I32) for _ in range(2)],
            [pltpu.VMEM((CH, D), F32) for _ in range(2)],
            [pltpu.VMEM((CH, ZW), F32) for _ in range(2)],
            [pltpu.SemaphoreType.DMA for _ in range(2)],
            [pltpu.SemaphoreType.DMA for _ in range(2)],
        ],
    )
    def k(y_hbm, r_hbm, pad_hbm, x_hbm, zaux_hbm, xs_hbm, zs_hbm, slot_hbm,
          pad_v, yv, rv, slotv, xv, zv, semx, semz):
        wid = lax.axis_index("s") * NC + lax.axis_index("c")
        base = wid * TPW
        pltpu.sync_copy(pad_hbm, pad_v)
        pltpu.sync_copy(y_hbm.at[pl.ds(base, TPW)], yv)
        pltpu.sync_copy(r_hbm.at[pl.ds(base, TPW)], rv)
        pv = pad_v[...]
        hx = [None, None]
        hz = [None, None]
        for c in range(NCH):
            b = c % 2
            t0 = base + c * CH
            if hx[b] is not None:
                hx[b].wait()
                hz[b].wait()
            for kk in range(CH // L):
                off = c * CH + kk * L
                y16 = yv[pl.ds(off, L)]
                r16 = rv[pl.ds(off, L)]
                po = jnp.zeros((L,), I32)
                for e in range(N_BR):
                    pe = lax.squeeze(lax.slice(pv, (e,), (e + 1,)), (0,))
                    po = jnp.where(y16 == e, pe, po)
                slotv[b][pl.ds(kk * L, L)] = po + r16
            pltpu.sync_copy(x_hbm.at[pl.ds(t0, CH)], xv[b])
            pltpu.sync_copy(zaux_hbm.at[pl.ds(t0, CH)], zv[b])
            hx[b] = pltpu.async_copy(xv[b], xs_hbm.at[slotv[b]], semx[b])
            hz[b] = pltpu.async_copy(zv[b], zs_hbm.at[slotv[b]], semz[b])
            pltpu.sync_copy(slotv[b], slot_hbm.at[pl.ds(t0, CH)])
        for b in range(2):
            if hx[b] is not None:
                hx[b].wait()
                hz[b].wait()

    return k(y_flat, rank_flat, pad16, x, zaux)


def _sc_combine(buf, slot):
    D = buf.shape[1]
    N = slot.shape[0]
    mesh = plsc.VectorSubcoreMesh(core_axis_name="c", subcore_axis_name="s")

    @functools.partial(
        pl.kernel, mesh=mesh,
        out_type=jax.ShapeDtypeStruct((N, D), F32),
        scratch_types=[
            pltpu.VMEM((TPW,), I32),
            [pltpu.VMEM((CH,), I32) for _ in range(2)],
            [pltpu.VMEM((CH, D), F32) for _ in range(2)],
            [pltpu.SemaphoreType.DMA for _ in range(2)],
        ],
    )
    def k(buf_hbm, slot_hbm, out_hbm, sv, svc, bv, sem):
        wid = lax.axis_index("s") * NC + lax.axis_index("c")
        base = wid * TPW
        pltpu.sync_copy(slot_hbm.at[pl.ds(base, TPW)], sv)

        def fill_and_fire(c):
            b = c % 2
            for kk in range(CH // L):
                svc[b][pl.ds(kk * L, L)] = sv[pl.ds(c * CH + kk * L, L)]
            return pltpu.async_copy(buf_hbm.at[svc[b]], bv[b], sem[b])

        h = [None, None]
        h[0] = fill_and_fire(0)
        for c in range(NCH):
            b = c % 2
            if c + 1 < NCH:
                h[1 - b] = fill_and_fire(c + 1)
            h[b].wait()
            pltpu.sync_copy(bv[b], out_hbm.at[pl.ds(base + c * CH, CH)])

    return k(buf, slot)


def _mlp_body(eot_ref, xs_ref, zs_ref, W1_ref, b1_ref, W2_ref, b2_ref, out_ref):
    xb = xs_ref[...]
    a = jnp.maximum(xb, 0.0)
    h = jnp.maximum(jnp.dot(a, W1_ref[0], preferred_element_type=F32) + b1_ref[0], 0.0)
    sub = jnp.dot(h, W2_ref[0], preferred_element_type=F32) + b2_ref[0]
    out_ref[...] = xb + sub * zs_ref[:, 0:1]


def _run_grouped_mlp(eot, xs, zs, W1, b1r, W2, b2r):
    D, H, E = N_DIMS, N_SM, N_BR
    grid_spec = pltpu.PrefetchScalarGridSpec(
        num_scalar_prefetch=1,
        grid=(NTILES,),
        in_specs=[
            pl.BlockSpec((TMB, D), lambda i, s: (i, 0)),
            pl.BlockSpec((TMB, ZW), lambda i, s: (i, 0)),
            pl.BlockSpec((1, D, H), lambda i, s: (s[i], 0, 0)),
            pl.BlockSpec((1, 1, H), lambda i, s: (s[i], 0, 0)),
            pl.BlockSpec((1, H, D), lambda i, s: (s[i], 0, 0)),
            pl.BlockSpec((1, 1, D), lambda i, s: (s[i], 0, 0)),
        ],
        out_specs=pl.BlockSpec((TMB, D), lambda i, s: (i, 0)),
    )
    fn = pl.pallas_call(
        _mlp_body,
        grid_spec=grid_spec,
        out_shape=jax.ShapeDtypeStruct((PAD, D), F32),
        compiler_params=pltpu.CompilerParams(dimension_semantics=("arbitrary",)),
    )
    return fn(eot, xs, zs, W1, b1r, W2, b2r)


def kernel(x, W_sw, b_sw, W1, b1, W2, b2, exp_noise, gauss_noise):
    N, D = x.shape
    E = exp_noise.shape[1]
    H = W1.shape[2]
    bsw2 = b_sw.reshape(3 * E, 1)
    b1r = b1.reshape(E, 1, H)
    b2r = b2.reshape(E, 1, D)

    ylogT, yidxT, yhardT, zmT, zlT, zgT, yidx1, rank1, zaux, pad16, eot = \
        _run_router(x, W_sw, bsw2, jnp.transpose(exp_noise),
                    jnp.transpose(gauss_noise))

    xs, zs, slot = _sc_dispatch(x, zaux, yidx1, rank1, pad16)
    buf = _run_grouped_mlp(eot, xs, zs, W1, b1r, W2, b2r)
    out1 = _sc_combine(buf, slot)

    return (out1, jnp.transpose(ylogT), jnp.transpose(yidxT),
            jnp.transpose(yhardT), jnp.transpose(zmT), jnp.transpose(zlT),
            jnp.transpose(zgT))


# speedup vs baseline: 1.1982x; 1.0316x over previous
import functools

import jax
import jax.numpy as jnp
from jax import lax
from jax.experimental import pallas as pl
from jax.experimental.pallas import tpu as pltpu
from jax.experimental.pallas import tpu_sc as plsc

F32 = jnp.float32
I32 = jnp.int32

N_TOK, N_DIMS, N_SM, N_BR = 8192, 2048, 1024, 8
TM = 512
TMB = 256
PAD = N_TOK + N_BR * TMB
NTILES = PAD // TMB
NC, NS, L = 2, 16, 16
NW = NC * NS
TPW = N_TOK // NW
CH = 16
ZW = 128
NCH = TPW // CH


def _router_body(x_ref, Wsw_ref, bsw_ref, enT_ref, gnT_ref,
                 yl_ref, yi_ref, yh_ref, zm_ref, zl_ref, zg_ref,
                 yi1_ref, rank_ref, zaux_ref, pad16_ref, eot_ref, nl_ref,
                 cnt_ref):
    i = pl.program_id(0)
    nt = pl.num_programs(0)

    @pl.when(i == 0)
    def _():
        cnt_ref[...] = jnp.zeros_like(cnt_ref)

    xb = x_ref[...]
    tm = xb.shape[0]
    E = enT_ref.shape[0]
    a = jnp.maximum(xb, 0.0)
    ctrlT = lax.dot_general(Wsw_ref[...], a, (((0,), (1,)), ((), ())),
                            preferred_element_type=F32) + bsw_ref[...]
    ylogT = ctrlT[0:E]
    zmeanT = ctrlT[E:2 * E]
    zlogvT = ctrlT[2 * E:3 * E]
    gT = -jnp.log(enT_ref[...] + 1e-20)
    glT = (ylogT + gT) / 1.0
    ysoftT = jax.nn.softmax(glT, axis=0)
    m = jnp.max(ysoftT, axis=0, keepdims=True)
    lanesT = lax.broadcasted_iota(I32, (E, tm), 0)
    cand = jnp.where(ysoftT == m, lanesT, E)
    yidxT = jnp.min(cand, axis=0, keepdims=True)
    onehotT = (lanesT == yidxT).astype(F32)
    yhardT = (onehotT - ysoftT) + ysoftT
    zT = gnT_ref[...] * jnp.exp(zlogvT / 2.0) + zmeanT

    yl_ref[...] = ylogT
    yi_ref[...] = yidxT
    yh_ref[...] = yhardT
    zm_ref[...] = jnp.sum(onehotT * zmeanT, axis=0, keepdims=True)
    zl_ref[...] = jnp.sum(onehotT * zlogvT, axis=0, keepdims=True)
    zgT = jnp.sum(onehotT * zT, axis=0, keepdims=True)
    zg_ref[...] = zgT
    zaux_ref[...] = jnp.broadcast_to(jnp.transpose(zgT), zaux_ref.shape)
    yi1_ref[...] = jnp.reshape(yidxT, (tm,))

    rows = lax.broadcasted_iota(I32, (tm, tm), 0)
    cols = lax.broadcasted_iota(I32, (tm, tm), 1)
    triu = (rows < cols).astype(F32)
    exclT = jnp.dot(onehotT, triu, preferred_element_type=F32)
    carry = cnt_ref[...]
    rank_ref[...] = jnp.sum(onehotT * (exclT + carry), axis=0).astype(I32)
    ones_col = jnp.ones((tm, 1), F32)
    new_cnt = carry + jnp.dot(onehotT, ones_col, preferred_element_type=F32)
    cnt_ref[...] = new_cnt

    @pl.when(i == nt - 1)
    def _():
        padded = jnp.floor((new_cnt + (TMB - 1)) * (1.0 / TMB)) * TMB
        tri8 = (lax.broadcasted_iota(I32, (E, E), 1)
                <= lax.broadcasted_iota(I32, (E, E), 0)).astype(F32)
        pad_end = jnp.dot(tri8, padded, preferred_element_type=F32)
        pad_off = pad_end - padded
        pad16_ref[...] = jnp.reshape(jnp.concatenate(
            (pad_off, jnp.zeros_like(pad_off)), axis=0).astype(I32), (2 * E,))
        starts = (lax.broadcasted_iota(I32, (1, NTILES), 1) * TMB).astype(F32)
        le = (jnp.broadcast_to(pad_end, (E, NTILES)) <= starts).astype(F32)
        eot_ref[...] = jnp.clip(jnp.sum(le, axis=0), 0.0, E - 1.0).astype(I32)
        nl_ref[...] = jnp.reshape(jnp.sum(padded) * (1.0 / TMB), (1,)).astype(I32)


def _run_router(x, W_sw, bsw2, exp_noise, gauss_noise):
    N, D = x.shape
    E = exp_noise.shape[0]
    nt = N // TM
    out_shapes = (
        jax.ShapeDtypeStruct((E, N), F32),
        jax.ShapeDtypeStruct((1, N), I32),
        jax.ShapeDtypeStruct((E, N), F32),
        jax.ShapeDtypeStruct((1, N), F32),
        jax.ShapeDtypeStruct((1, N), F32),
        jax.ShapeDtypeStruct((1, N), F32),
        jax.ShapeDtypeStruct((N,), I32),
        jax.ShapeDtypeStruct((N,), I32),
        jax.ShapeDtypeStruct((N, ZW), F32),
        jax.ShapeDtypeStruct((2 * E,), I32),
        jax.ShapeDtypeStruct((NTILES,), I32),
        jax.ShapeDtypeStruct((1,), I32),
    )
    fn = pl.pallas_call(
        _router_body,
        grid=(nt,),
        in_specs=[
            pl.BlockSpec((TM, D), lambda i: (i, 0)),
            pl.BlockSpec((D, 3 * E), lambda i: (0, 0)),
            pl.BlockSpec((3 * E, 1), lambda i: (0, 0)),
            pl.BlockSpec((E, TM), lambda i: (0, i)),
            pl.BlockSpec((E, TM), lambda i: (0, i)),
        ],
        out_specs=(
            pl.BlockSpec((E, TM), lambda i: (0, i)),
            pl.BlockSpec((1, TM), lambda i: (0, i)),
            pl.BlockSpec((E, TM), lambda i: (0, i)),
            pl.BlockSpec((1, TM), lambda i: (0, i)),
            pl.BlockSpec((1, TM), lambda i: (0, i)),
            pl.BlockSpec((1, TM), lambda i: (0, i)),
            pl.BlockSpec((TM,), lambda i: (i,)),
            pl.BlockSpec((TM,), lambda i: (i,)),
            pl.BlockSpec((TM, ZW), lambda i: (i, 0)),
            pl.BlockSpec((2 * E,), lambda i: (0,)),
            pl.BlockSpec((NTILES,), lambda i: (0,)),
            pl.BlockSpec((1,), lambda i: (0,)),
        ),
        out_shape=out_shapes,
        scratch_shapes=[pltpu.VMEM((E, 1), F32)],
        compiler_params=pltpu.CompilerParams(dimension_semantics=("arbitrary",)),
    )
    return fn(x, W_sw, bsw2, exp_noise, gauss_noise)


def _sc_dispatch(x, zaux, y_flat, rank_flat, pad16):
    N, D = x.shape
    mesh = plsc.VectorSubcoreMesh(core_axis_name="c", subcore_axis_name="s")

    @functools.partial(
        pl.kernel, mesh=mesh,
        out_type=(
            jax.ShapeDtypeStruct((PAD, D), F32),
            jax.ShapeDtypeStruct((PAD, ZW), F32),
            jax.ShapeDtypeStruct((N,), I32),
        ),
        scratch_types=[
            pltpu.VMEM((L,), I32),
            pltpu.VMEM((TPW,), I32),
            pltpu.VMEM((TPW,), I32),
            [pltpu.VMEM((CH,), I32) for _ in range(2)],
            [pltpu.VMEM((CH, D), F32) for _ in range(2)],
            [pltpu.VMEM((CH, ZW), F32) for _ in range(2)],
            [pltpu.SemaphoreType.DMA for _ in range(2)],
            [pltpu.SemaphoreType.DMA for _ in range(2)],
        ],
    )
    def k(y_hbm, r_hbm, pad_hbm, x_hbm, zaux_hbm, xs_hbm, zs_hbm, slot_hbm,
          pad_v, yv, rv, slotv, xv, zv, semx, semz):
        wid = lax.axis_index("s") * NC + lax.axis_index("c")
        base = wid * TPW
        pltpu.sync_copy(pad_hbm, pad_v)
        pltpu.sync_copy(y_hbm.at[pl.ds(base, TPW)], yv)
        pltpu.sync_copy(r_hbm.at[pl.ds(base, TPW)], rv)
        pv = pad_v[...]
        hx = [None, None]
        hz = [None, None]
        for c in range(NCH):
            b = c % 2
            t0 = base + c * CH
            if hx[b] is not None:
                hx[b].wait()
                hz[b].wait()
            for kk in range(CH // L):
                off = c * CH + kk * L
                y16 = yv[pl.ds(off, L)]
                r16 = rv[pl.ds(off, L)]
                po = jnp.zeros((L,), I32)
                for e in range(N_BR):
                    pe = lax.squeeze(lax.slice(pv, (e,), (e + 1,)), (0,))
                    po = jnp.where(y16 == e, pe, po)
                slotv[b][pl.ds(kk * L, L)] = po + r16
            pltpu.sync_copy(x_hbm.at[pl.ds(t0, CH)], xv[b])
            pltpu.sync_copy(zaux_hbm.at[pl.ds(t0, CH)], zv[b])
            hx[b] = pltpu.async_copy(xv[b], xs_hbm.at[slotv[b]], semx[b])
            hz[b] = pltpu.async_copy(zv[b], zs_hbm.at[slotv[b]], semz[b])
            pltpu.sync_copy(slotv[b], slot_hbm.at[pl.ds(t0, CH)])
        for b in range(2):
            if hx[b] is not None:
                hx[b].wait()
                hz[b].wait()

    return k(y_flat, rank_flat, pad16, x, zaux)


def _sc_combine(buf, slot):
    D = buf.shape[1]
    N = slot.shape[0]
    mesh = plsc.VectorSubcoreMesh(core_axis_name="c", subcore_axis_name="s")

    @functools.partial(
        pl.kernel, mesh=mesh,
        out_type=jax.ShapeDtypeStruct((N, D), F32),
        scratch_types=[
            pltpu.VMEM((TPW,), I32),
            [pltpu.VMEM((CH,), I32) for _ in range(2)],
            [pltpu.VMEM((CH, D), F32) for _ in range(2)],
            [pltpu.SemaphoreType.DMA for _ in range(2)],
        ],
    )
    def k(buf_hbm, slot_hbm, out_hbm, sv, svc, bv, sem):
        wid = lax.axis_index("s") * NC + lax.axis_index("c")
        base = wid * TPW
        pltpu.sync_copy(slot_hbm.at[pl.ds(base, TPW)], sv)

        def fill_and_fire(c):
            b = c % 2
            for kk in range(CH // L):
                svc[b][pl.ds(kk * L, L)] = sv[pl.ds(c * CH + kk * L, L)]
            return pltpu.async_copy(buf_hbm.at[svc[b]], bv[b], sem[b])

        h = [None, None]
        h[0] = fill_and_fire(0)
        for c in range(NCH):
            b = c % 2
            if c + 1 < NCH:
                h[1 - b] = fill_and_fire(c + 1)
            h[b].wait()
            pltpu.sync_copy(bv[b], out_hbm.at[pl.ds(base + c * CH, CH)])

    return k(buf, slot)


def _mlp_body(eot_ref, nl_ref, xs_ref, zs_ref, W1_ref, b1_ref, W2_ref, b2_ref,
              out_ref):
    @pl.when(pl.program_id(0) < nl_ref[0])
    def _():
        xb = xs_ref[...]
        a = jnp.maximum(xb, 0.0)
        h = jnp.maximum(jnp.dot(a, W1_ref[0], preferred_element_type=F32) + b1_ref[0], 0.0)
        sub = jnp.dot(h, W2_ref[0], preferred_element_type=F32) + b2_ref[0]
        out_ref[...] = xb + sub * zs_ref[:, 0:1]


def _run_grouped_mlp(eot, nlive, xs, zs, W1, b1r, W2, b2r):
    D, H, E = N_DIMS, N_SM, N_BR

    def _j(i, nl):
        return jnp.minimum(i, nl[0] - 1)

    grid_spec = pltpu.PrefetchScalarGridSpec(
        num_scalar_prefetch=2,
        grid=(NTILES,),
        in_specs=[
            pl.BlockSpec((TMB, D), lambda i, s, nl: (_j(i, nl), 0)),
            pl.BlockSpec((TMB, ZW), lambda i, s, nl: (_j(i, nl), 0)),
            pl.BlockSpec((1, D, H), lambda i, s, nl: (s[_j(i, nl)], 0, 0)),
            pl.BlockSpec((1, 1, H), lambda i, s, nl: (s[_j(i, nl)], 0, 0)),
            pl.BlockSpec((1, H, D), lambda i, s, nl: (s[_j(i, nl)], 0, 0)),
            pl.BlockSpec((1, 1, D), lambda i, s, nl: (s[_j(i, nl)], 0, 0)),
        ],
        out_specs=pl.BlockSpec((TMB, D), lambda i, s, nl: (_j(i, nl), 0)),
    )
    fn = pl.pallas_call(
        _mlp_body,
        grid_spec=grid_spec,
        out_shape=jax.ShapeDtypeStruct((PAD, D), F32),
        compiler_params=pltpu.CompilerParams(dimension_semantics=("arbitrary",)),
    )
    return fn(eot, nlive, xs, zs, W1, b1r, W2, b2r)


def kernel(x, W_sw, b_sw, W1, b1, W2, b2, exp_noise, gauss_noise):
    N, D = x.shape
    E = exp_noise.shape[1]
    H = W1.shape[2]
    bsw2 = b_sw.reshape(3 * E, 1)
    b1r = b1.reshape(E, 1, H)
    b2r = b2.reshape(E, 1, D)

    (ylogT, yidxT, yhardT, zmT, zlT, zgT, yidx1, rank1, zaux, pad16, eot,
     nlive) = _run_router(x, W_sw, bsw2, jnp.transpose(exp_noise),
                          jnp.transpose(gauss_noise))

    xs, zs, slot = _sc_dispatch(x, zaux, yidx1, rank1, pad16)
    buf = _run_grouped_mlp(eot, nlive, xs, zs, W1, b1r, W2, b2r)
    out1 = _sc_combine(buf, slot)

    return (out1, jnp.transpose(ylogT), jnp.transpose(yidxT),
            jnp.transpose(yhardT), jnp.transpose(zmT), jnp.transpose(zlT),
            jnp.transpose(zgT))
